# baseline (speedup 1.0000x reference)
import functools

import numpy as np

import jax
import jax.numpy as jnp
from jax import lax
from jax.experimental import pallas as pl
from jax.experimental.pallas import tpu as pltpu
from jax.experimental.pallas import tpu_sc as plsc

_B = 16384
_L = 200
_VOCAB = 7800
_DIM = 64

_NC = 2
_NS = 16
_NW = _NC * _NS
_LANES = 16
_ROWS_PER_W = _B // _NW
_GROUPS = _ROWS_PER_W // _LANES


def _scores_body(table_ref, w_ref, s_ref):
    s_ref[...] = jnp.dot(table_ref[...], w_ref[...].reshape(_DIM),
                         preferred_element_type=jnp.float32)


def _vocab_scores(embed_table, lin_w):
    return pl.pallas_call(
        _scores_body,
        out_shape=jax.ShapeDtypeStruct((_VOCAB,), jnp.float32),
    )(embed_table, lin_w)


def _sc_pool(x, length, scores, bias16):
    mesh = plsc.VectorSubcoreMesh(core_axis_name="c", subcore_axis_name="s")

    @functools.partial(
        pl.kernel,
        mesh=mesh,
        compiler_params=pltpu.CompilerParams(needs_layout_passes=False),
        out_type=jax.ShapeDtypeStruct((_B,), jnp.float32),
        scratch_types=[
            pltpu.VMEM((_VOCAB,), jnp.float32),
            pltpu.VMEM((2 * _LANES, _L), jnp.int32),
            pltpu.VMEM((_ROWS_PER_W,), jnp.float32),
            pltpu.VMEM((_LANES,), jnp.float32),
            pltpu.VMEM((_ROWS_PER_W,), jnp.float32),
            pltpu.SemaphoreType.DMA,
            pltpu.SemaphoreType.DMA,
        ],
    )
    def pool(x_hbm, len_hbm, s_hbm, b_hbm, out_hbm,
             s_v, xt, len_v, b_v, out_v, sem0, sem1):
        sems = (sem0, sem1)
        wid = lax.axis_index("s") * _NC + lax.axis_index("c")
        row0 = wid * _ROWS_PER_W

        def issue(g, buf):
            pltpu.async_copy(
                x_hbm.at[pl.ds(row0 + g * _LANES, _LANES), :],
                xt.at[pl.ds(buf * _LANES, _LANES), :], sems[buf])

        issue(jnp.int32(0), 0)
        issue(jnp.int32(1), 1)
        pltpu.sync_copy(s_hbm, s_v)
        pltpu.sync_copy(len_hbm.at[pl.ds(row0, _ROWS_PER_W)], len_v)
        pltpu.sync_copy(b_hbm, b_v)
        bias = b_v[...]
        zeros = jnp.zeros((_LANES,), jnp.float32)

        lane = lax.iota(jnp.int32, _LANES)
        rvecs = [lane + (b * _LANES) for b in (0, 1)]

        def pair(i, carry):
            for buf in (0, 1):
                g = 2 * i + buf
                pltpu.make_async_copy(
                    x_hbm.at[pl.ds(0, _LANES), :],
                    xt.at[pl.ds(buf * _LANES, _LANES), :], sems[buf]).wait()

                accs = [zeros] * 4
                for l in range(_L):
                    col = jnp.full((_LANES,), l, jnp.int32)
                    xi = plsc.load_gather(xt, [rvecs[buf], col])
                    accs[l % 4] = accs[l % 4] + plsc.load_gather(s_v, [xi])
                acc = (accs[0] + accs[1]) + (accs[2] + accs[3])
                sl = pl.ds(g * _LANES, _LANES)
                t = acc / len_v[sl] + bias
                out_v[sl] = 1.0 / (1.0 + jnp.exp(-t))

                @pl.when(g + 2 < _GROUPS)
                def _prefetch():
                    issue(g + 2, buf)
            return carry

        lax.fori_loop(0, _GROUPS // 2, pair, 0)
        pltpu.sync_copy(out_v, out_hbm.at[pl.ds(row0, _ROWS_PER_W)])

    return pool(x, length, scores, bias16)


@jax.jit
def kernel(x, length, embed_table, lin_w, lin_b):
    scores = _vocab_scores(embed_table, lin_w)
    bias16 = jnp.broadcast_to(lin_b.astype(jnp.float32), (_LANES,))
    y = _sc_pool(x, length, scores, bias16)
    return y.reshape(_B, 1)


# speedup vs baseline: 1.6642x; 1.6642x over previous
import functools

import jax
import jax.numpy as jnp
from jax import lax
from jax.experimental import pallas as pl
from jax.experimental.pallas import tpu as pltpu
from jax.experimental.pallas import tpu_sc as plsc

_B = 16384
_L = 200
_VOCAB = 7800
_DIM = 64

_NC = 2
_NS = 16
_NW = _NC * _NS
_LANES = 16
_ROWS_PER_W = _B // _NW
_GROUPS = _ROWS_PER_W // _LANES
_FULL = (_L // _LANES) * _LANES
_CHUNKS = list(range(0, _FULL, _LANES)) + [_L - _LANES]


def _scores_body(table_ref, w_ref, s_ref):
    s_ref[...] = jnp.dot(table_ref[...], w_ref[...].reshape(_DIM),
                         preferred_element_type=jnp.float32)


def _vocab_scores(embed_table, lin_w):
    return pl.pallas_call(
        _scores_body,
        out_shape=jax.ShapeDtypeStruct((_VOCAB,), jnp.float32),
    )(embed_table, lin_w)


def _sc_pool(x, length, scores, bias16):
    mesh = plsc.VectorSubcoreMesh(core_axis_name="c", subcore_axis_name="s")

    @functools.partial(
        pl.kernel,
        mesh=mesh,
        compiler_params=pltpu.CompilerParams(needs_layout_passes=False),
        out_type=jax.ShapeDtypeStruct((_B,), jnp.float32),
        scratch_types=[
            pltpu.VMEM((_VOCAB,), jnp.float32),
            pltpu.VMEM((2 * _LANES, _L), jnp.int32),
            pltpu.VMEM((_ROWS_PER_W,), jnp.float32),
            pltpu.VMEM((_LANES,), jnp.float32),
            pltpu.VMEM((_ROWS_PER_W,), jnp.float32),
            pltpu.SemaphoreType.DMA,
        ],
    )
    def pool(x_hbm, len_hbm, s_hbm, b_hbm, out_hbm,
             s_v, xt, len_v, b_v, out_v, sem):
        wid = lax.axis_index("s") * _NC + lax.axis_index("c")
        row0 = wid * _ROWS_PER_W
        lane = lax.iota(jnp.int32, _LANES)
        tail_keep = lane >= (_LANES - (_L - _FULL))
        zeros = jnp.zeros((_LANES,), jnp.float32)

        def issue(g, buf):
            pltpu.async_copy(
                x_hbm.at[pl.ds(row0 + g * _LANES, _LANES), :],
                xt.at[pl.ds(buf * _LANES, _LANES), :], sem)

        issue(jnp.int32(0), jnp.int32(0))
        pltpu.sync_copy(s_hbm, s_v)
        pltpu.sync_copy(len_hbm.at[pl.ds(row0, _ROWS_PER_W)], len_v)
        pltpu.sync_copy(b_hbm, b_v)

        last = lane == (_LANES - 1)

        def row_sum(r_local, r_global):
            acc0, acc1 = zeros, zeros
            for k, c in enumerate(_CHUNKS):
                xi = xt[r_local, pl.ds(c, _LANES)]
                sc = plsc.load_gather(s_v, [xi])
                if c == _CHUNKS[-1]:
                    sc = jnp.where(tail_keep, sc, zeros)
                if k % 2 == 0:
                    acc0 = acc0 + sc
                else:
                    acc1 = acc1 + sc
            cum = plsc.cumsum(acc0 + acc1)
            plsc.store_scatter(out_v, [jnp.full((_LANES,), r_global)], cum,
                               mask=last)

        def group(g, carry):
            buf = g % 2
            pltpu.make_async_copy(
                x_hbm.at[pl.ds(0, _LANES), :],
                xt.at[pl.ds(buf * _LANES, _LANES), :], sem).wait()

            @pl.when(g + 1 < _GROUPS)
            def _prefetch():
                issue(g + 1, (g + 1) % 2)

            for r in range(_LANES):
                row_sum(buf * _LANES + r, g * _LANES + r)
            return carry

        lax.fori_loop(0, _GROUPS, group, 0)

        bias = b_v[...]

        def finish(k, carry):
            sl = pl.ds(k * _LANES, _LANES)
            t = out_v[sl] / len_v[sl] + bias
            out_v[sl] = 1.0 / (1.0 + jnp.exp(-t))
            return carry

        lax.fori_loop(0, _GROUPS, finish, 0)
        pltpu.sync_copy(out_v, out_hbm.at[pl.ds(row0, _ROWS_PER_W)])

    return pool(x, length, scores, bias16)


@jax.jit
def kernel(x, length, embed_table, lin_w, lin_b):
    scores = _vocab_scores(embed_table, lin_w)
    bias16 = jnp.broadcast_to(lin_b.astype(jnp.float32), (_LANES,))
    y = _sc_pool(x, length, scores, bias16)
    return y.reshape(_B, 1)


# speedup vs baseline: 1.7567x; 1.0555x over previous
import functools

import jax
import jax.numpy as jnp
from jax import lax
from jax.experimental import pallas as pl
from jax.experimental.pallas import tpu as pltpu
from jax.experimental.pallas import tpu_sc as plsc

_B = 16384
_L = 200
_VOCAB = 7800
_DIM = 64

_NC = 2
_NS = 16
_NW = _NC * _NS
_LANES = 16
_ROWS_PER_W = _B // _NW
_GROUPS = _ROWS_PER_W // _LANES
_FULL = (_L // _LANES) * _LANES
_CHUNKS = list(range(0, _FULL, _LANES)) + [_L - _LANES]


def _scores_body(table_ref, w_ref, s_ref):
    s_ref[...] = jnp.dot(table_ref[...], w_ref[...].reshape(_DIM),
                         preferred_element_type=jnp.float32)


def _vocab_scores(embed_table, lin_w):
    return pl.pallas_call(
        _scores_body,
        out_shape=jax.ShapeDtypeStruct((_VOCAB,), jnp.float32),
    )(embed_table, lin_w)


def _sc_pool(x, length, scores, bias16):
    mesh = plsc.VectorSubcoreMesh(core_axis_name="c", subcore_axis_name="s")

    @functools.partial(
        pl.kernel,
        mesh=mesh,
        compiler_params=pltpu.CompilerParams(needs_layout_passes=False),
        out_type=jax.ShapeDtypeStruct((_B,), jnp.float32),
        scratch_types=[
            pltpu.VMEM((_VOCAB,), jnp.float32),
            pltpu.VMEM((2 * _LANES, _L), jnp.int32),
            pltpu.VMEM((_ROWS_PER_W,), jnp.float32),
            pltpu.VMEM((_LANES,), jnp.float32),
            pltpu.VMEM((_ROWS_PER_W,), jnp.float32),
            pltpu.SemaphoreType.DMA,
            pltpu.SemaphoreType.DMA,
        ],
    )
    def pool(x_hbm, len_hbm, s_hbm, b_hbm, out_hbm,
             s_v, xt, len_v, b_v, out_v, sem0, sem1):
        sems = (sem0, sem1)
        wid = lax.axis_index("s") * _NC + lax.axis_index("c")
        row0 = wid * _ROWS_PER_W
        lane = lax.iota(jnp.int32, _LANES)
        tail_keep = lane >= (_LANES - (_L - _FULL))
        zeros = jnp.zeros((_LANES,), jnp.float32)

        def issue(g, buf):
            pltpu.async_copy(
                x_hbm.at[pl.ds(row0 + g * _LANES, _LANES), :],
                xt.at[pl.ds(buf * _LANES, _LANES), :], sems[buf])

        issue(jnp.int32(0), 0)
        issue(jnp.int32(1), 1)
        pltpu.sync_copy(s_hbm, s_v)
        pltpu.sync_copy(len_hbm.at[pl.ds(row0, _ROWS_PER_W)], len_v)
        pltpu.sync_copy(b_hbm, b_v)

        last = lane == (_LANES - 1)

        def row_sum(r_local, r_global):
            acc0, acc1 = zeros, zeros
            for k, c in enumerate(_CHUNKS):
                xi = xt[r_local, pl.ds(c, _LANES)]
                sc = plsc.load_gather(s_v, [xi])
                if c == _CHUNKS[-1]:
                    sc = jnp.where(tail_keep, sc, zeros)
                if k % 2 == 0:
                    acc0 = acc0 + sc
                else:
                    acc1 = acc1 + sc
            cum = plsc.cumsum(acc0 + acc1)
            plsc.store_scatter(out_v, [jnp.full((_LANES,), r_global)], cum,
                               mask=last)

        def pair(i, carry):
            for buf in (0, 1):
                g = 2 * i + buf
                pltpu.make_async_copy(
                    x_hbm.at[pl.ds(0, _LANES), :],
                    xt.at[pl.ds(buf * _LANES, _LANES), :], sems[buf]).wait()

                for r in range(_LANES):
                    row_sum(buf * _LANES + r, g * _LANES + r)

                @pl.when(g + 2 < _GROUPS)
                def _prefetch():
                    issue(g + 2, buf)
            return carry

        lax.fori_loop(0, _GROUPS // 2, pair, 0)

        bias = b_v[...]

        def finish(k, carry):
            sl = pl.ds(k * _LANES, _LANES)
            t = out_v[sl] / len_v[sl] + bias
            out_v[sl] = 1.0 / (1.0 + jnp.exp(-t))
            return carry

        lax.fori_loop(0, _GROUPS, finish, 0)
        pltpu.sync_copy(out_v, out_hbm.at[pl.ds(row0, _ROWS_PER_W)])

    return pool(x, length, scores, bias16)


@jax.jit
def kernel(x, length, embed_table, lin_w, lin_b):
    scores = _vocab_scores(embed_table, lin_w)
    bias16 = jnp.broadcast_to(lin_b.astype(jnp.float32), (_LANES,))
    y = _sc_pool(x, length, scores, bias16)
    return y.reshape(_B, 1)
